# Initial kernel scaffold; baseline (speedup 1.0000x reference)
import jax
import jax.numpy as jnp
from jax.experimental import pallas as pl


def kernel(sequences, segments, token_table, pos_table, seg_table):
    raise NotImplementedError("write your pallas kernel here")


# speedup vs baseline: 1.5929x; 1.5929x over previous
import functools

import jax
import jax.numpy as jnp
from jax import lax
from jax.experimental import pallas as pl
from jax.experimental.pallas import tpu as pltpu
from jax.experimental.pallas import tpu_sc as plsc

VOCAB = 1000000
MAX_LEN = 512
DIM = 64
B = 1024
S = 512

NC = 2
NS = 16
NW = NC * NS
ROWS = B * S
RPW = ROWS // NW
C = 128
NCHUNK = RPW // C


def _body(seq_hbm, seg_hbm, tok_hbm, pos_hbm, segtab_hbm, out_hbm,
          base_v, segtab_v, idx_v, sgv_v, buf_v, sem):
    cid = lax.axis_index("c")
    sid = lax.axis_index("s")
    wid = sid * NC + cid

    pltpu.sync_copy(pos_hbm, base_v.at[pl.ds(0, S), :])
    pltpu.sync_copy(pos_hbm, base_v.at[pl.ds(S, S), :])
    pltpu.sync_copy(segtab_hbm, segtab_v)

    seg_rows = [[segtab_v[k, pl.ds(j * 16, 16)] for j in range(4)]
                for k in range(2)]

    def build(r, carry):
        for j in range(4):
            sl = pl.ds(j * 16, 16)
            plsc.addupdate(base_v.at[r, sl], seg_rows[0][j])
            plsc.addupdate(base_v.at[S + r, sl], seg_rows[1][j])
        return carry

    lax.fori_loop(0, S, build, 0)

    row0 = wid * RPW

    def chunk_body(c, carry):
        base = row0 + c * C
        pltpu.sync_copy(seq_hbm.at[pl.ds(base, C)], idx_v)
        pltpu.sync_copy(seg_hbm.at[pl.ds(base, C)], sgv_v)
        pltpu.async_copy(tok_hbm.at[idx_v], buf_v, sem).wait()

        def row_body(g, rcarry):
            sgvec = sgv_v[pl.ds(g * 16, 16)]
            for r in range(16):
                i = g * 16 + r
                s = lax.rem(base + i, S)
                brow = sgvec[r] * S + s
                for j in range(4):
                    sl = pl.ds(j * 16, 16)
                    plsc.addupdate(buf_v.at[i, sl], base_v[brow, sl])
            return rcarry

        lax.fori_loop(0, C // 16, row_body, 0)
        pltpu.sync_copy(buf_v, out_hbm.at[pl.ds(base, C), :])
        return carry

    lax.fori_loop(0, NCHUNK, chunk_body, 0)


@jax.jit
def _run(seq_flat, seg_flat, token_table, pos_table, seg_table):
    mesh = plsc.VectorSubcoreMesh(core_axis_name="c", subcore_axis_name="s")
    f = functools.partial(
        pl.kernel,
        out_type=jax.ShapeDtypeStruct((ROWS, DIM), jnp.float32),
        mesh=mesh,
        scratch_types=[
            pltpu.VMEM((2 * S, DIM), jnp.float32),
            pltpu.VMEM((2, DIM), jnp.float32),
            pltpu.VMEM((C,), jnp.int32),
            pltpu.VMEM((C,), jnp.int32),
            pltpu.VMEM((C, DIM), jnp.float32),
            pltpu.SemaphoreType.DMA,
        ],
        compiler_params=pltpu.CompilerParams(use_tc_tiling_on_sc=False),
    )(_body)
    return f(seq_flat, seg_flat, token_table, pos_table, seg_table)


def kernel(sequences, segments, token_table, pos_table, seg_table):
    seq_flat = sequences.reshape(ROWS).astype(jnp.int32)
    seg_flat = segments.reshape(ROWS).astype(jnp.int32)
    out = _run(seq_flat, seg_flat, token_table, pos_table, seg_table)
    return out.reshape(B, S, DIM)


# speedup vs baseline: 1.9845x; 1.2458x over previous
import functools

import jax
import jax.numpy as jnp
from jax import lax
from jax.experimental import pallas as pl
from jax.experimental.pallas import tpu as pltpu
from jax.experimental.pallas import tpu_sc as plsc

VOCAB = 1000000
MAX_LEN = 512
DIM = 64
B = 1024
S = 512

NC = 2
NS = 16
NW = NC * NS
ROWS = B * S
RPW = ROWS // NW
C = 128
NCHUNK = RPW // C
NBUF = 4


def _body(seq_hbm, seg_hbm, tok_hbm, pos_hbm, segtab_hbm, out_hbm,
          base_v, segtab_v, idx_v, sgv_v, buf_v,
          gsem, ssem, isem, msem):
    cid = lax.axis_index("c")
    sid = lax.axis_index("s")
    wid = sid * NC + cid

    pltpu.sync_copy(pos_hbm, base_v.at[pl.ds(0, S), :])
    pltpu.sync_copy(pos_hbm, base_v.at[pl.ds(S, S), :])
    pltpu.sync_copy(segtab_hbm, segtab_v)

    seg_rows = [[segtab_v[k, pl.ds(j * 16, 16)] for j in range(4)]
                for k in range(2)]

    def build(r, carry):
        for j in range(4):
            sl = pl.ds(j * 16, 16)
            plsc.addupdate(base_v.at[r, sl], seg_rows[0][j])
            plsc.addupdate(base_v.at[S + r, sl], seg_rows[1][j])
        return carry

    lax.fori_loop(0, S, build, 0)

    row0 = wid * RPW
    lanes = lax.iota(jnp.int32, 16)

    def idx_copies(c, b):
        base = row0 + c * C
        return (
            pltpu.make_async_copy(seq_hbm.at[pl.ds(base, C)], idx_v.at[b],
                                  isem.at[b]),
            pltpu.make_async_copy(seg_hbm.at[pl.ds(base, C)], sgv_v.at[b],
                                  msem.at[b]),
        )

    def gather_copy(b):
        return pltpu.make_async_copy(tok_hbm.at[idx_v.at[b]], buf_v.at[b],
                                     gsem.at[b])

    def scatter_copy(c, b):
        base = row0 + c * C
        return pltpu.make_async_copy(buf_v.at[b],
                                     out_hbm.at[pl.ds(base, C), :], ssem.at[b])

    for b in range(2):
        ci, cs = idx_copies(b, b)
        ci.start()
        cs.start()
    ic, sc_ = idx_copies(0, 0)
    ic.wait()
    sc_.wait()
    gather_copy(0).start()

    def outer(t, carry):
        for b in range(NBUF):
            c = t * NBUF + b
            gather_copy(b).wait()
            @pl.when(c + 2 < NCHUNK)
            def _():
                ci, cs = idx_copies(c + 2, (c + 2) % NBUF)
                ci.start()
                cs.start()
            bn = (b + 1) % NBUF

            @pl.when(c + 1 < NCHUNK)
            def _():
                ci, cs = idx_copies(c + 1, bn)
                ci.wait()
                cs.wait()

                @pl.when(c >= 3)
                def _():
                    scatter_copy(c - 3, bn).wait()

                gather_copy(bn).start()

            m0 = lax.rem(c * C, S)

            def group(g, gcarry):
                sgvec = sgv_v[b, pl.ds(g * 16, 16)]
                brows = sgvec * S + (m0 + g * 16) + lanes
                for r in range(16):
                    i = g * 16 + r
                    br = brows[r]
                    for j in range(4):
                        sl = pl.ds(j * 16, 16)
                        plsc.addupdate(buf_v.at[b, i, sl], base_v[br, sl])
                return gcarry

            lax.fori_loop(0, C // 16, group, 0)
            scatter_copy(c, b).start()
        return carry

    lax.fori_loop(0, NCHUNK // NBUF, outer, 0)

    for b in range(NBUF):
        c = NCHUNK - NBUF + b
        scatter_copy(c, b).wait()


@jax.jit
def _run(seq_flat, seg_flat, token_table, pos_table, seg_table):
    mesh = plsc.VectorSubcoreMesh(core_axis_name="c", subcore_axis_name="s")
    f = functools.partial(
        pl.kernel,
        out_type=jax.ShapeDtypeStruct((ROWS, DIM), jnp.float32),
        mesh=mesh,
        scratch_types=[
            pltpu.VMEM((2 * S, DIM), jnp.float32),
            pltpu.VMEM((2, DIM), jnp.float32),
            pltpu.VMEM((NBUF, C), jnp.int32),
            pltpu.VMEM((NBUF, C), jnp.int32),
            pltpu.VMEM((NBUF, C, DIM), jnp.float32),
            pltpu.SemaphoreType.DMA((NBUF,)),
            pltpu.SemaphoreType.DMA((NBUF,)),
            pltpu.SemaphoreType.DMA((NBUF,)),
            pltpu.SemaphoreType.DMA((NBUF,)),
        ],
        compiler_params=pltpu.CompilerParams(use_tc_tiling_on_sc=False),
    )(_body)
    return f(seq_flat, seg_flat, token_table, pos_table, seg_table)


def kernel(sequences, segments, token_table, pos_table, seg_table):
    seq_flat = sequences.reshape(ROWS).astype(jnp.int32)
    seg_flat = segments.reshape(ROWS).astype(jnp.int32)
    out = _run(seq_flat, seg_flat, token_table, pos_table, seg_table)
    return out.reshape(B, S, DIM)
